# baseline (speedup 1.0000x reference)
import jax
import jax.numpy as jnp
from jax.experimental import pallas as pl


def _assign_kernel(x_ref, c_ref, out_ref):
    x = x_ref[...]
    c = c_ref[...]
    cnorm = jnp.sum(c * c, axis=0, keepdims=True)
    xnorm = jnp.sum(x * x, axis=1, keepdims=True)
    dist = xnorm - 2.0 * jnp.dot(x, c, preferred_element_type=jnp.float32) + cnorm
    out_ref[...] = jnp.argmin(dist, axis=-1).astype(jnp.int32)


def kernel(x, centroids):
    n, d = x.shape
    _, k = centroids.shape
    bn = 256
    return pl.pallas_call(
        _assign_kernel,
        grid=(n // bn,),
        in_specs=[
            pl.BlockSpec((bn, d), lambda i: (i, 0)),
            pl.BlockSpec((d, k), lambda i: (0, 0)),
        ],
        out_specs=pl.BlockSpec((bn,), lambda i: (i,)),
        out_shape=jax.ShapeDtypeStruct((n,), jnp.int32),
    )(x, centroids)


# speedup vs baseline: 1.5936x; 1.5936x over previous
import jax
import jax.numpy as jnp
from jax.experimental import pallas as pl
from jax.experimental.pallas import tpu as pltpu


def _assign_kernel(x_ref, c_ref, out_ref, cnorm_ref):
    @pl.when(pl.program_id(0) == 0)
    def _():
        c = c_ref[...]
        cnorm_ref[...] = jnp.sum(c * c, axis=0, keepdims=True)

    x = x_ref[...]
    xnorm = jnp.sum(x * x, axis=1, keepdims=True)
    m2 = jnp.dot(x + x, c_ref[...], preferred_element_type=jnp.float32)
    dist = xnorm - m2 + cnorm_ref[...]
    out_ref[...] = jnp.argmin(dist, axis=-1).astype(jnp.int32)


def kernel(x, centroids):
    n, d = x.shape
    _, k = centroids.shape
    bn = 512
    return pl.pallas_call(
        _assign_kernel,
        grid=(n // bn,),
        in_specs=[
            pl.BlockSpec((bn, d), lambda i: (i, 0)),
            pl.BlockSpec((d, k), lambda i: (0, 0)),
        ],
        out_specs=pl.BlockSpec((bn,), lambda i: (i,)),
        out_shape=jax.ShapeDtypeStruct((n,), jnp.int32),
        scratch_shapes=[pltpu.VMEM((1, k), jnp.float32)],
    )(x, centroids)


# speedup vs baseline: 1.8217x; 1.1432x over previous
import jax
import jax.numpy as jnp
from jax.experimental import pallas as pl
from jax.experimental.pallas import tpu as pltpu


def _assign_kernel(x_ref, c_ref, out_ref, ct2_ref, cn_ref):
    @pl.when(pl.program_id(0) == 0)
    def _():
        c = c_ref[...]
        ct = jnp.transpose(c, (1, 0))
        ct2_ref[...] = ct + ct
        cn_row = jnp.sum(c * c, axis=0, keepdims=True)
        cn_ref[...] = jnp.transpose(cn_row, (1, 0))

    x = x_ref[...]
    xn = jnp.sum(x * x, axis=1, keepdims=True)
    xnr = jnp.transpose(xn, (1, 0))
    mt2 = jax.lax.dot_general(ct2_ref[...], x,
                              (((1,), (1,)), ((), ())),
                              preferred_element_type=jnp.float32)
    key = (xnr - mt2) + cn_ref[...]
    out_ref[...] = jnp.argmin(key, axis=0).astype(jnp.int32)


def kernel(x, centroids):
    n, d = x.shape
    _, k = centroids.shape
    bn = 512
    return pl.pallas_call(
        _assign_kernel,
        grid=(n // bn,),
        in_specs=[
            pl.BlockSpec((bn, d), lambda i: (i, 0)),
            pl.BlockSpec((d, k), lambda i: (0, 0)),
        ],
        out_specs=pl.BlockSpec((bn,), lambda i: (i,)),
        out_shape=jax.ShapeDtypeStruct((n,), jnp.int32),
        scratch_shapes=[pltpu.VMEM((k, d), jnp.float32),
                        pltpu.VMEM((k, 1), jnp.float32)],
    )(x, centroids)


# speedup vs baseline: 2.8067x; 1.5406x over previous
import jax
import jax.numpy as jnp
from jax.experimental import pallas as pl
from jax.experimental.pallas import tpu as pltpu

_BN = 512
_DN = (((1,), (1,)), ((), ()))


def _assign_kernel(x1, x2, x3, x4, c_ref, oe_ref, oo_ref, ct2_ref, cn_ref,
                   sa, sb, sc, sd):
    @pl.when(pl.program_id(0) == 0)
    def _():
        c = c_ref[...]
        ct = jnp.transpose(c, (1, 0))
        ct2_ref[...] = ct + ct
        cn_row = jnp.sum(c * c, axis=0, keepdims=True)
        cn_ref[...] = jnp.transpose(cn_row, (1, 0))

    ct2 = ct2_ref[...]
    cn = cn_ref[...]
    sa[...] = jax.lax.dot_general(ct2, x1[...], _DN,
                                  preferred_element_type=jnp.float32)
    sb[...] = jax.lax.dot_general(ct2, x2[...], _DN,
                                  preferred_element_type=jnp.float32)
    r1 = jnp.argmin(cn - sc[...], axis=0).astype(jnp.int32)
    r2 = jnp.argmin(cn - sd[...], axis=0).astype(jnp.int32)
    oo_ref[...] = jnp.concatenate([r1, r2]).reshape(1, 1, 2 * _BN)
    r3 = jnp.argmin(cn - sa[...], axis=0).astype(jnp.int32)
    r4 = jnp.argmin(cn - sb[...], axis=0).astype(jnp.int32)
    oe_ref[...] = jnp.concatenate([r3, r4]).reshape(1, 1, 2 * _BN)
    sc[...] = jax.lax.dot_general(ct2, x3[...], _DN,
                                  preferred_element_type=jnp.float32)
    sd[...] = jax.lax.dot_general(ct2, x4[...], _DN,
                                  preferred_element_type=jnp.float32)


def kernel(x, centroids):
    n, d = x.shape
    _, k = centroids.shape
    bn = _BN
    nblk = n // bn
    npair = nblk // 2
    grid = nblk // 4 + 1
    last = nblk - 1
    xspec = lambda off: pl.BlockSpec(
        (bn, d), lambda i, off=off: (jnp.minimum(4 * i + off, last), 0))
    oe, oo = pl.pallas_call(
        _assign_kernel,
        grid=(grid,),
        in_specs=[xspec(0), xspec(1), xspec(2), xspec(3),
                  pl.BlockSpec((d, k), lambda i: (0, 0))],
        out_specs=[
            pl.BlockSpec((1, 1, 2 * bn),
                         lambda i: (jnp.minimum(i, grid - 1), 0, 0)),
            pl.BlockSpec((1, 1, 2 * bn),
                         lambda i: (jnp.maximum(i - 1, 0), 0, 0)),
        ],
        out_shape=[jax.ShapeDtypeStruct((grid, 1, 2 * bn), jnp.int32),
                   jax.ShapeDtypeStruct((npair // 2, 1, 2 * bn), jnp.int32)],
        scratch_shapes=[pltpu.VMEM((k, d), jnp.float32),
                        pltpu.VMEM((k, 1), jnp.float32),
                        pltpu.VMEM((k, bn), jnp.float32),
                        pltpu.VMEM((k, bn), jnp.float32),
                        pltpu.VMEM((k, bn), jnp.float32),
                        pltpu.VMEM((k, bn), jnp.float32)],
    )(x, x, x, x, centroids)
    even = oe[: npair // 2, 0, :]
    odd = oo[:, 0, :]
    return jnp.stack([even, odd], axis=1).reshape(-1)
